# baseline (speedup 1.0000x reference)
import functools

import jax
import jax.numpy as jnp
from jax import lax
from jax.experimental import pallas as pl
from jax.experimental.pallas import tpu as pltpu
from jax.experimental.pallas import tpu_sc as plsc

_N = 10000
_E = 320000
_D = 128

_NC = 2
_NS = 16
_NW = _NC * _NS
_EPW = _E // _NW
_CHUNK = 640
_NPAD = _NS * _CHUNK

_ROWS = 1024


def _sc_count_body(idx_hbm, ones_hbm, out_hbm, idx_v, ones_v, shared):
    cid = lax.axis_index("c")
    sid = lax.axis_index("s")
    wid = sid * _NC + cid

    pltpu.sync_copy(ones_hbm, ones_v)
    pltpu.sync_copy(idx_hbm.at[pl.ds(wid * _EPW, _EPW)], idx_v)
    pltpu.sync_copy(ones_v.at[pl.ds(_EPW, _CHUNK)],
                    shared.at[pl.ds(sid * _CHUNK, _CHUNK)])
    plsc.subcore_barrier()
    pltpu.sync_copy(ones_v.at[pl.ds(0, _EPW)], shared.at[idx_v], add=True)
    plsc.subcore_barrier()
    pltpu.sync_copy(
        shared.at[pl.ds(sid * _CHUNK, _CHUNK)],
        out_hbm.at[pl.ds(cid * _NPAD + sid * _CHUNK, _CHUNK)],
    )


@functools.cache
def _sc_count():
    return pl.kernel(
        _sc_count_body,
        out_type=jax.ShapeDtypeStruct((_NC * _NPAD,), jnp.float32),
        mesh=plsc.VectorSubcoreMesh(core_axis_name="c", subcore_axis_name="s",
                                    num_cores=_NC, num_subcores=_NS),
        scratch_types=[
            pltpu.VMEM((_EPW,), jnp.int32),
            pltpu.VMEM((_EPW + _CHUNK,), jnp.float32),
            pltpu.VMEM_SHARED((_NPAD,), jnp.float32),
        ],
    )


def _layernorm(x, w, b):
    m = jnp.mean(x, axis=-1, keepdims=True)
    var = jnp.mean((x - m) * (x - m), axis=-1, keepdims=True)
    return (x - m) * lax.rsqrt(var + 1e-5) * w + b


def _dot_nt(x, w):
    return lax.dot_general(x, w, (((1,), (1,)), ((), ())),
                           preferred_element_type=jnp.float32)


def _tc_body(v_ref, c0_ref, c1_ref, wv_ref, bv_ref, wo_ref, bo_ref,
             ln1w_ref, ln1b_ref, w1_ref, b1_ref, w2_ref, b2_ref,
             ln2w_ref, ln2b_ref, out_ref):
    xv = v_ref[:]
    ct = (c0_ref[:] + c1_ref[:]).T
    cnt = jnp.concatenate(
        [lax.slice(ct, (0, a), (_D, a + 1)) for a in range(_ROWS // _D)],
        axis=0)
    ind = jnp.where(cnt > 0.0, 1.0, 0.0)
    vl = _dot_nt(xv, wv_ref[:]) + bv_ref[:]
    agg = vl * ind
    v2 = _dot_nt(agg, wo_ref[:]) + bo_ref[:]
    x = _layernorm(xv + v2, ln1w_ref[:], ln1b_ref[:])
    h = jnp.maximum(_dot_nt(x, w1_ref[:]) + b1_ref[:], 0.0)
    v2 = _dot_nt(h, w2_ref[:]) + b2_ref[:]
    out_ref[:] = _layernorm(x + v2, ln2w_ref[:], ln2b_ref[:])


def _row_block(i):
    return (i, 0)


def _pinned(i):
    return (0, 0)


_GRID = _NPAD // _ROWS


def _c1_block(i):
    return (i + _GRID, 0)


_tc_fused = pl.pallas_call(
    _tc_body,
    grid=(_GRID,),
    in_specs=[
        pl.BlockSpec((_ROWS, _D), _row_block),
        pl.BlockSpec((_ROWS // _D, _D), _row_block),
        pl.BlockSpec((_ROWS // _D, _D), _c1_block),
        pl.BlockSpec((_D, _D), _pinned),
        pl.BlockSpec((1, _D), _pinned),
        pl.BlockSpec((_D, _D), _pinned),
        pl.BlockSpec((1, _D), _pinned),
        pl.BlockSpec((1, _D), _pinned),
        pl.BlockSpec((1, _D), _pinned),
        pl.BlockSpec((_D, _D), _pinned),
        pl.BlockSpec((1, _D), _pinned),
        pl.BlockSpec((_D, _D), _pinned),
        pl.BlockSpec((1, _D), _pinned),
        pl.BlockSpec((1, _D), _pinned),
        pl.BlockSpec((1, _D), _pinned),
    ],
    out_specs=pl.BlockSpec((_ROWS, _D), _row_block),
    out_shape=jax.ShapeDtypeStruct((_N, _D), jnp.float32),
    compiler_params=pltpu.CompilerParams(
        dimension_semantics=("parallel",),
    ),
)


def kernel(q, k, v, self_indices, neighbor_indices, Wq, bq, Wk, bk, Wv, bv,
           Wo, bo, ln1_w, ln1_b, W1, b1, W2, b2, ln2_w, ln2_b):
    payload = jnp.concatenate([jnp.ones((_EPW,), jnp.float32),
                               jnp.zeros((_CHUNK,), jnp.float32)])
    counts = _sc_count()(self_indices, payload)
    counts = counts.reshape(_NC * _NPAD // _D, _D)
    row = lambda a: a.reshape(1, _D)
    return _tc_fused(
        v, counts, counts,
        Wv, row(bv), Wo, row(bo),
        row(ln1_w), row(ln1_b),
        W1, row(b1), W2, row(b2),
        row(ln2_w), row(ln2_b),
    )


# speedup vs baseline: 1.0892x; 1.0892x over previous
import functools

import jax
import jax.numpy as jnp
from jax import lax
from jax.experimental import pallas as pl
from jax.experimental.pallas import tpu as pltpu
from jax.experimental.pallas import tpu_sc as plsc

_N = 10000
_E = 320000
_D = 128

_NC = 2
_NS = 16
_NW = _NC * _NS
_EPW = _E // _NW
_CHUNK = 640
_NPAD = _NS * _CHUNK

_ROWS = 2048


def _sc_count_body(idx_hbm, ones_hbm, out_hbm, idx_v, ones_v, shared):
    cid = lax.axis_index("c")
    sid = lax.axis_index("s")
    wid = sid * _NC + cid

    pltpu.sync_copy(ones_hbm, ones_v)
    pltpu.sync_copy(idx_hbm.at[pl.ds(wid * _EPW, _EPW)], idx_v)
    pltpu.sync_copy(ones_v.at[pl.ds(_EPW, _CHUNK)],
                    shared.at[pl.ds(sid * _CHUNK, _CHUNK)])
    plsc.subcore_barrier()
    pltpu.sync_copy(ones_v.at[pl.ds(0, _EPW)], shared.at[idx_v], add=True)
    plsc.subcore_barrier()
    pltpu.sync_copy(
        shared.at[pl.ds(sid * _CHUNK, _CHUNK)],
        out_hbm.at[pl.ds(cid * _NPAD + sid * _CHUNK, _CHUNK)],
    )


@functools.cache
def _sc_count():
    return pl.kernel(
        _sc_count_body,
        out_type=jax.ShapeDtypeStruct((_NC * _NPAD,), jnp.float32),
        mesh=plsc.VectorSubcoreMesh(core_axis_name="c", subcore_axis_name="s",
                                    num_cores=_NC, num_subcores=_NS),
        scratch_types=[
            pltpu.VMEM((_EPW,), jnp.int32),
            pltpu.VMEM((_EPW + _CHUNK,), jnp.float32),
            pltpu.VMEM_SHARED((_NPAD,), jnp.float32),
        ],
    )


def _layernorm(x, w, b):
    m = jnp.mean(x, axis=-1, keepdims=True)
    var = jnp.mean((x - m) * (x - m), axis=-1, keepdims=True)
    return (x - m) * lax.rsqrt(var + 1e-5) * w + b


def _dot_nt(x, w):
    return lax.dot_general(x, w, (((1,), (1,)), ((), ())),
                           preferred_element_type=jnp.float32)


def _tc_body(v_ref, c0_ref, c1_ref, wv_ref, bv_ref, wo_ref, bo_ref,
             ln1w_ref, ln1b_ref, w1_ref, b1_ref, w2_ref, b2_ref,
             ln2w_ref, ln2b_ref, out_ref):
    xv = v_ref[:]
    ct = (c0_ref[:] + c1_ref[:]).T
    cnt = jnp.concatenate(
        [lax.slice(ct, (0, a), (_D, a + 1)) for a in range(_ROWS // _D)],
        axis=0)
    ind = jnp.where(cnt > 0.0, 1.0, 0.0)
    vl = _dot_nt(xv, wv_ref[:]) + bv_ref[:]
    agg = vl * ind
    v2 = _dot_nt(agg, wo_ref[:]) + bo_ref[:]
    x = _layernorm(xv + v2, ln1w_ref[:], ln1b_ref[:])
    h = jnp.maximum(_dot_nt(x, w1_ref[:]) + b1_ref[:], 0.0)
    v2 = _dot_nt(h, w2_ref[:]) + b2_ref[:]
    out_ref[:] = _layernorm(x + v2, ln2w_ref[:], ln2b_ref[:])


def _row_block(i):
    return (i, 0)


def _pinned(i):
    return (0, 0)


_GRID = _NPAD // _ROWS


def _c1_block(i):
    return (i + _GRID, 0)


_tc_fused = pl.pallas_call(
    _tc_body,
    grid=(_GRID,),
    in_specs=[
        pl.BlockSpec((_ROWS, _D), _row_block),
        pl.BlockSpec((_ROWS // _D, _D), _row_block),
        pl.BlockSpec((_ROWS // _D, _D), _c1_block),
        pl.BlockSpec((_D, _D), _pinned),
        pl.BlockSpec((1, _D), _pinned),
        pl.BlockSpec((_D, _D), _pinned),
        pl.BlockSpec((1, _D), _pinned),
        pl.BlockSpec((1, _D), _pinned),
        pl.BlockSpec((1, _D), _pinned),
        pl.BlockSpec((_D, _D), _pinned),
        pl.BlockSpec((1, _D), _pinned),
        pl.BlockSpec((_D, _D), _pinned),
        pl.BlockSpec((1, _D), _pinned),
        pl.BlockSpec((1, _D), _pinned),
        pl.BlockSpec((1, _D), _pinned),
    ],
    out_specs=pl.BlockSpec((_ROWS, _D), _row_block),
    out_shape=jax.ShapeDtypeStruct((_N, _D), jnp.float32),
    compiler_params=pltpu.CompilerParams(
        dimension_semantics=("parallel",),
    ),
)


def kernel(q, k, v, self_indices, neighbor_indices, Wq, bq, Wk, bk, Wv, bv,
           Wo, bo, ln1_w, ln1_b, W1, b1, W2, b2, ln2_w, ln2_b):
    payload = jnp.concatenate([jnp.ones((_EPW,), jnp.float32),
                               jnp.zeros((_CHUNK,), jnp.float32)])
    counts = _sc_count()(self_indices, payload)
    counts = counts.reshape(_NC * _NPAD // _D, _D)
    row = lambda a: a.reshape(1, _D)
    return _tc_fused(
        v, counts, counts,
        Wv, row(bv), Wo, row(bo),
        row(ln1_w), row(ln1_b),
        W1, row(b1), W2, row(b2),
        row(ln2_w), row(ln2_b),
    )


# speedup vs baseline: 1.3347x; 1.2254x over previous
import functools

import jax
import jax.numpy as jnp
from jax import lax
from jax.experimental import pallas as pl
from jax.experimental.pallas import tpu as pltpu
from jax.experimental.pallas import tpu_sc as plsc

_N = 10000
_E = 320000
_D = 128

_NC = 2
_NS = 16
_NW = _NC * _NS
_EPW = _E // _NW
_CHUNK = 640
_NPAD = _NS * _CHUNK
_PAD0 = _N - (_NS - 1) * _CHUNK

_ROWS = 2048
_GRID = _NPAD // _ROWS
_TAIL = _N - (_GRID - 1) * _ROWS


def _sc_count_body(idx_hbm, pay_hbm, out_hbm, idx_v, pay_v, shared):
    cid = lax.axis_index("c")
    sid = lax.axis_index("s")
    wid = sid * _NC + cid

    pltpu.sync_copy(pay_hbm, pay_v)
    pltpu.sync_copy(idx_hbm.at[pl.ds(wid * _EPW, _EPW)], idx_v)
    init_off = jnp.where(sid == _NS - 1, _EPW + _CHUNK, _EPW)
    pltpu.sync_copy(pay_v.at[pl.ds(init_off, _CHUNK)],
                    shared.at[pl.ds(sid * _CHUNK, _CHUNK)])
    plsc.subcore_barrier()
    pltpu.sync_copy(pay_v.at[pl.ds(0, _EPW)], shared.at[idx_v], add=True)
    plsc.subcore_barrier()
    pltpu.sync_copy(
        shared.at[pl.ds(sid * _CHUNK, _CHUNK)],
        out_hbm.at[pl.ds(cid * _NPAD + sid * _CHUNK, _CHUNK)],
    )


@functools.cache
def _sc_count():
    return pl.kernel(
        _sc_count_body,
        out_type=jax.ShapeDtypeStruct((_NC * _NPAD,), jnp.float32),
        mesh=plsc.VectorSubcoreMesh(core_axis_name="c", subcore_axis_name="s",
                                    num_cores=_NC, num_subcores=_NS),
        scratch_types=[
            pltpu.VMEM((_EPW,), jnp.int32),
            pltpu.VMEM((_EPW + 2 * _CHUNK,), jnp.float32),
            pltpu.VMEM_SHARED((_NPAD,), jnp.float32),
        ],
    )


def _layernorm(x, w, b):
    m = jnp.mean(x, axis=-1, keepdims=True)
    var = jnp.mean((x - m) * (x - m), axis=-1, keepdims=True)
    return (x - m) * lax.rsqrt(var + 1e-5) * w + b


def _dot_nt(x, w):
    return lax.dot_general(x, w, (((1,), (1,)), ((), ())),
                           preferred_element_type=jnp.float32)


def _dense_pipeline(xv, ind, wv, bv, wo, bo, ln1w, ln1b, w1, b1, w2, b2,
                    ln2w, ln2b):
    vl = _dot_nt(xv, wv) + bv
    if ind is not None:
        vl = vl * ind
    v2 = _dot_nt(vl, wo) + bo
    x = _layernorm(xv + v2, ln1w, ln1b)
    h = jnp.maximum(_dot_nt(x, w1) + b1, 0.0)
    v2 = _dot_nt(h, w2) + b2
    return _layernorm(x + v2, ln2w, ln2b)


def _tc_main_body(v_ref, wv_ref, bv_ref, wo_ref, bo_ref, ln1w_ref, ln1b_ref,
                  w1_ref, b1_ref, w2_ref, b2_ref, ln2w_ref, ln2b_ref,
                  out_ref):
    out_ref[:] = _dense_pipeline(
        v_ref[:], None, wv_ref[:], bv_ref[:], wo_ref[:], bo_ref[:],
        ln1w_ref[:], ln1b_ref[:], w1_ref[:], b1_ref[:], w2_ref[:], b2_ref[:],
        ln2w_ref[:], ln2b_ref[:])


def _row_block(i):
    return (i, 0)


def _pinned(i):
    return (0, 0)


def _c1_block(i):
    return (i + _GRID, 0)


_W_SPECS = [
    pl.BlockSpec((_D, _D), _pinned),
    pl.BlockSpec((1, _D), _pinned),
    pl.BlockSpec((_D, _D), _pinned),
    pl.BlockSpec((1, _D), _pinned),
    pl.BlockSpec((1, _D), _pinned),
    pl.BlockSpec((1, _D), _pinned),
    pl.BlockSpec((_D, _D), _pinned),
    pl.BlockSpec((1, _D), _pinned),
    pl.BlockSpec((_D, _D), _pinned),
    pl.BlockSpec((1, _D), _pinned),
    pl.BlockSpec((1, _D), _pinned),
    pl.BlockSpec((1, _D), _pinned),
]

_tc_main = pl.pallas_call(
    _tc_main_body,
    grid=(_GRID,),
    in_specs=[pl.BlockSpec((_ROWS, _D), _row_block)] + _W_SPECS,
    out_specs=pl.BlockSpec((_ROWS, _D), _row_block),
    out_shape=jax.ShapeDtypeStruct((_N, _D), jnp.float32),
    compiler_params=pltpu.CompilerParams(
        dimension_semantics=("parallel",),
    ),
)


def _lane_to_rows(c, nrows):
    ct = c.T
    return jnp.concatenate(
        [lax.slice(ct, (0, a), (_D, a + 1)) for a in range(nrows // _D)],
        axis=0)


def _tc_fix_body(c0_ref, c1_ref, wv_ref, bv_ref, wo_ref, bo_ref,
                 ln1w_ref, ln1b_ref, w1_ref, b1_ref, w2_ref, b2_ref,
                 ln2w_ref, ln2b_ref, v_any, outin_any, out_any, xv_scr, sem):
    i = pl.program_id(0)
    c = c0_ref[:] + c1_ref[:]
    has_empty = jnp.any(c <= 0.5)

    weights = (wv_ref[:], bv_ref[:], wo_ref[:], bo_ref[:], ln1w_ref[:],
               ln1b_ref[:], w1_ref[:], b1_ref[:], w2_ref[:], b2_ref[:],
               ln2w_ref[:], ln2b_ref[:])

    def redo(nrows):
        cp = pltpu.make_async_copy(
            v_any.at[pl.ds(i * _ROWS, nrows), :],
            xv_scr.at[pl.ds(0, nrows), :], sem)
        cp.start()
        cp.wait()
        cnt = _lane_to_rows(c, _ROWS)[:nrows]
        ind = jnp.where(cnt > 0.5, 1.0, 0.0)
        xv = xv_scr[pl.ds(0, nrows), :]
        xv_scr[pl.ds(0, nrows), :] = _dense_pipeline(xv, ind, *weights)
        cp = pltpu.make_async_copy(
            xv_scr.at[pl.ds(0, nrows), :],
            out_any.at[pl.ds(i * _ROWS, nrows), :], sem)
        cp.start()
        cp.wait()

    @pl.when(jnp.logical_and(has_empty, i < _GRID - 1))
    def _():
        redo(_ROWS)

    @pl.when(jnp.logical_and(has_empty, i == _GRID - 1))
    def _():
        redo(_TAIL)


_tc_fix = pl.pallas_call(
    _tc_fix_body,
    grid=(_GRID,),
    in_specs=[
        pl.BlockSpec((_ROWS // _D, _D), _row_block),
        pl.BlockSpec((_ROWS // _D, _D), _c1_block),
    ] + _W_SPECS + [
        pl.BlockSpec(memory_space=pl.ANY),
        pl.BlockSpec(memory_space=pl.ANY),
    ],
    out_specs=pl.BlockSpec(memory_space=pl.ANY),
    out_shape=jax.ShapeDtypeStruct((_N, _D), jnp.float32),
    scratch_shapes=[
        pltpu.VMEM((_ROWS, _D), jnp.float32),
        pltpu.SemaphoreType.DMA,
    ],
    input_output_aliases={15: 0},
    compiler_params=pltpu.CompilerParams(
        dimension_semantics=("arbitrary",),
    ),
)


def kernel(q, k, v, self_indices, neighbor_indices, Wq, bq, Wk, bk, Wv, bv,
           Wo, bo, ln1_w, ln1_b, W1, b1, W2, b2, ln2_w, ln2_b):
    payload = jnp.concatenate([
        jnp.ones((_EPW,), jnp.float32),
        jnp.zeros((_CHUNK,), jnp.float32),
        jnp.concatenate([jnp.zeros((_PAD0,), jnp.float32),
                         jnp.ones((_CHUNK - _PAD0,), jnp.float32)]),
    ])
    counts = _sc_count()(self_indices, payload)
    counts = counts.reshape(_NC * _NPAD // _D, _D)
    row = lambda a: a.reshape(1, _D)
    weights = (Wv, row(bv), Wo, row(bo), row(ln1_w), row(ln1_b),
               W1, row(b1), W2, row(b2), row(ln2_w), row(ln2_b))
    main = _tc_main(v, *weights)
    return _tc_fix(counts, counts, *weights, v, main)


# speedup vs baseline: 1.3978x; 1.0473x over previous
import functools

import jax
import jax.numpy as jnp
from jax import lax
from jax.experimental import pallas as pl
from jax.experimental.pallas import tpu as pltpu
from jax.experimental.pallas import tpu_sc as plsc

_N = 10000
_E = 320000
_D = 128

_NC = 2
_NS = 16
_NW = _NC * _NS
_EPW = _E // _NW
_CHUNK = 640
_NPAD = _NS * _CHUNK
_PAD0 = _N - (_NS - 1) * _CHUNK

_ROWS = 2048
_GRID = _NPAD // _ROWS
_TAIL = _N - (_GRID - 1) * _ROWS


def _sc_count_body(idx_hbm, pay_hbm, out_hbm, idx_v, pay_v, shared):
    cid = lax.axis_index("c")
    sid = lax.axis_index("s")
    wid = sid * _NC + cid

    pltpu.sync_copy(pay_hbm, pay_v)
    pltpu.sync_copy(idx_hbm.at[pl.ds(wid * _EPW, _EPW)], idx_v)
    init_off = jnp.where(sid == _NS - 1, _EPW + _CHUNK, _EPW)
    pltpu.sync_copy(pay_v.at[pl.ds(init_off, _CHUNK)],
                    shared.at[pl.ds(sid * _CHUNK, _CHUNK)])
    plsc.subcore_barrier()
    pltpu.sync_copy(pay_v.at[pl.ds(0, _EPW)], shared.at[idx_v], add=True)
    plsc.subcore_barrier()
    pltpu.sync_copy(
        shared.at[pl.ds(sid * _CHUNK, _CHUNK)],
        out_hbm.at[pl.ds(cid * _NPAD + sid * _CHUNK, _CHUNK)],
    )


@functools.cache
def _sc_count():
    return pl.kernel(
        _sc_count_body,
        out_type=jax.ShapeDtypeStruct((_NC * _NPAD,), jnp.float32),
        mesh=plsc.VectorSubcoreMesh(core_axis_name="c", subcore_axis_name="s",
                                    num_cores=_NC, num_subcores=_NS),
        scratch_types=[
            pltpu.VMEM((_EPW,), jnp.int32),
            pltpu.VMEM((_EPW + 2 * _CHUNK,), jnp.float32),
            pltpu.VMEM_SHARED((_NPAD,), jnp.float32),
        ],
    )


def _layernorm(x, w, b):
    m = jnp.mean(x, axis=-1, keepdims=True)
    var = jnp.mean((x - m) * (x - m), axis=-1, keepdims=True)
    return (x - m) * lax.rsqrt(var + 1e-5) * w + b


def _dot_nt(x, w):
    return lax.dot_general(x, w, (((1,), (1,)), ((), ())),
                           preferred_element_type=jnp.float32)


def _dense_pipeline(xv, ind, wv, bv, wo, bo, ln1w, ln1b, w1, b1, w2, b2,
                    ln2w, ln2b):
    vl = _dot_nt(xv, wv) + bv
    if ind is not None:
        vl = vl * ind
    v2 = _dot_nt(vl, wo) + bo
    x = _layernorm(xv + v2, ln1w, ln1b)
    h = jnp.maximum(_dot_nt(x, w1) + b1, 0.0)
    v2 = _dot_nt(h, w2) + b2
    return _layernorm(x + v2, ln2w, ln2b)


def _tc_main_body(v_ref, wv_ref, bv_ref, wo_ref, bo_ref, ln1w_ref, ln1b_ref,
                  w1_ref, b1_ref, w2_ref, b2_ref, ln2w_ref, ln2b_ref,
                  out_ref):
    out_ref[:] = _dense_pipeline(
        v_ref[:], None, wv_ref[:], bv_ref[:], wo_ref[:], bo_ref[:],
        ln1w_ref[:], ln1b_ref[:], w1_ref[:], b1_ref[:], w2_ref[:], b2_ref[:],
        ln2w_ref[:], ln2b_ref[:])


def _row_block(i):
    return (i, 0)


def _pinned(i):
    return (0, 0)


def _c1_block(i):
    return (i + _GRID, 0)


_W_SPECS = [
    pl.BlockSpec((_D, _D), _pinned),
    pl.BlockSpec((1, _D), _pinned),
    pl.BlockSpec((_D, _D), _pinned),
    pl.BlockSpec((1, _D), _pinned),
    pl.BlockSpec((1, _D), _pinned),
    pl.BlockSpec((1, _D), _pinned),
    pl.BlockSpec((_D, _D), _pinned),
    pl.BlockSpec((1, _D), _pinned),
    pl.BlockSpec((_D, _D), _pinned),
    pl.BlockSpec((1, _D), _pinned),
    pl.BlockSpec((1, _D), _pinned),
    pl.BlockSpec((1, _D), _pinned),
]

_tc_main = pl.pallas_call(
    _tc_main_body,
    grid=(_GRID,),
    in_specs=[pl.BlockSpec((_ROWS, _D), _row_block)] + _W_SPECS,
    out_specs=pl.BlockSpec((_ROWS, _D), _row_block),
    out_shape=jax.ShapeDtypeStruct((_N, _D), jnp.float32),
    compiler_params=pltpu.CompilerParams(
        dimension_semantics=("parallel",),
    ),
)


def _lane_to_rows(c, nrows):
    ct = c.T
    return jnp.concatenate(
        [lax.slice(ct, (0, a), (_D, a + 1)) for a in range(nrows // _D)],
        axis=0)


def _tc_fix_body(c_ref, wv_ref, bv_ref, wo_ref, bo_ref,
                 ln1w_ref, ln1b_ref, w1_ref, b1_ref, w2_ref, b2_ref,
                 ln2w_ref, ln2b_ref, v_any, outin_any, out_any, xv_scr, sem):
    cr = _NPAD // _D
    c = c_ref[0:cr, :] + c_ref[cr:2 * cr, :]

    weights = (wv_ref[:], bv_ref[:], wo_ref[:], bo_ref[:], ln1w_ref[:],
               ln1b_ref[:], w1_ref[:], b1_ref[:], w2_ref[:], b2_ref[:],
               ln2w_ref[:], ln2b_ref[:])

    rpb = _ROWS // _D
    for j in range(_GRID):
        cj = lax.slice(c, (j * rpb, 0), ((j + 1) * rpb, _D))
        nrows = _ROWS if j < _GRID - 1 else _TAIL

        @pl.when(jnp.any(cj <= 0.5))
        def _(cj=cj, j=j, nrows=nrows):
            cp = pltpu.make_async_copy(
                v_any.at[pl.ds(j * _ROWS, nrows), :],
                xv_scr.at[pl.ds(0, nrows), :], sem)
            cp.start()
            cp.wait()
            cnt = _lane_to_rows(cj, _ROWS)[:nrows]
            ind = jnp.where(cnt > 0.5, 1.0, 0.0)
            xv = xv_scr[pl.ds(0, nrows), :]
            xv_scr[pl.ds(0, nrows), :] = _dense_pipeline(xv, ind, *weights)
            cp = pltpu.make_async_copy(
                xv_scr.at[pl.ds(0, nrows), :],
                out_any.at[pl.ds(j * _ROWS, nrows), :], sem)
            cp.start()
            cp.wait()


_tc_fix = pl.pallas_call(
    _tc_fix_body,
    grid=(1,),
    in_specs=[
        pl.BlockSpec((_NC * _NPAD // _D, _D), _pinned),
    ] + _W_SPECS + [
        pl.BlockSpec(memory_space=pl.ANY),
        pl.BlockSpec(memory_space=pl.ANY),
    ],
    out_specs=pl.BlockSpec(memory_space=pl.ANY),
    out_shape=jax.ShapeDtypeStruct((_N, _D), jnp.float32),
    scratch_shapes=[
        pltpu.VMEM((_ROWS, _D), jnp.float32),
        pltpu.SemaphoreType.DMA,
    ],
    input_output_aliases={14: 0},
    compiler_params=pltpu.CompilerParams(
        dimension_semantics=("arbitrary",),
    ),
)


def kernel(q, k, v, self_indices, neighbor_indices, Wq, bq, Wk, bk, Wv, bv,
           Wo, bo, ln1_w, ln1_b, W1, b1, W2, b2, ln2_w, ln2_b):
    payload = jnp.concatenate([
        jnp.ones((_EPW,), jnp.float32),
        jnp.zeros((_CHUNK,), jnp.float32),
        jnp.concatenate([jnp.zeros((_PAD0,), jnp.float32),
                         jnp.ones((_CHUNK - _PAD0,), jnp.float32)]),
    ])
    counts = _sc_count()(self_indices, payload)
    counts = counts.reshape(_NC * _NPAD // _D, _D)
    row = lambda a: a.reshape(1, _D)
    weights = (Wv, row(bv), Wo, row(bo), row(ln1_w), row(ln1_b),
               W1, row(b1), W2, row(b2), row(ln2_w), row(ln2_b))
    main = _tc_main(v, *weights)
    return _tc_fix(counts, *weights, v, main)


# speedup vs baseline: 1.4006x; 1.0020x over previous
import functools

import jax
import jax.numpy as jnp
from jax import lax
from jax.experimental import pallas as pl
from jax.experimental.pallas import tpu as pltpu
from jax.experimental.pallas import tpu_sc as plsc

_N = 10000
_E = 320000
_D = 128

_NC = 2
_NS = 16
_NW = _NC * _NS
_EPW = _E // _NW
_CHUNK = 640
_NPAD = _NS * _CHUNK
_PAD0 = _N - (_NS - 1) * _CHUNK

_ROWS = 2560
_GRID = _NPAD // _ROWS
_TAIL = _N - (_GRID - 1) * _ROWS


def _sc_count_body(idx_hbm, pay_hbm, out_hbm, idx_v, pay_v, shared):
    cid = lax.axis_index("c")
    sid = lax.axis_index("s")
    wid = sid * _NC + cid

    pltpu.sync_copy(pay_hbm, pay_v)
    pltpu.sync_copy(idx_hbm.at[pl.ds(wid * _EPW, _EPW)], idx_v)
    init_off = jnp.where(sid == _NS - 1, _EPW + _CHUNK, _EPW)
    pltpu.sync_copy(pay_v.at[pl.ds(init_off, _CHUNK)],
                    shared.at[pl.ds(sid * _CHUNK, _CHUNK)])
    plsc.subcore_barrier()
    pltpu.sync_copy(pay_v.at[pl.ds(0, _EPW)], shared.at[idx_v], add=True)
    plsc.subcore_barrier()
    pltpu.sync_copy(
        shared.at[pl.ds(sid * _CHUNK, _CHUNK)],
        out_hbm.at[pl.ds(cid * _NPAD + sid * _CHUNK, _CHUNK)],
    )


@functools.cache
def _sc_count():
    return pl.kernel(
        _sc_count_body,
        out_type=jax.ShapeDtypeStruct((_NC * _NPAD,), jnp.float32),
        mesh=plsc.VectorSubcoreMesh(core_axis_name="c", subcore_axis_name="s",
                                    num_cores=_NC, num_subcores=_NS),
        scratch_types=[
            pltpu.VMEM((_EPW,), jnp.int32),
            pltpu.VMEM((_EPW + 2 * _CHUNK,), jnp.float32),
            pltpu.VMEM_SHARED((_NPAD,), jnp.float32),
        ],
    )


def _layernorm(x, w, b):
    m = jnp.mean(x, axis=-1, keepdims=True)
    var = jnp.mean((x - m) * (x - m), axis=-1, keepdims=True)
    return (x - m) * lax.rsqrt(var + 1e-5) * w + b


def _dot_nt(x, w):
    return lax.dot_general(x, w, (((1,), (1,)), ((), ())),
                           preferred_element_type=jnp.float32)


def _dense_pipeline(xv, ind, wv, bv, wo, bo, ln1w, ln1b, w1, b1, w2, b2,
                    ln2w, ln2b):
    vl = _dot_nt(xv, wv) + bv
    if ind is not None:
        vl = vl * ind
    v2 = _dot_nt(vl, wo) + bo
    x = _layernorm(xv + v2, ln1w, ln1b)
    h = jnp.maximum(_dot_nt(x, w1) + b1, 0.0)
    v2 = _dot_nt(h, w2) + b2
    return _layernorm(x + v2, ln2w, ln2b)


def _tc_main_body(v_ref, wv_ref, bv_ref, wo_ref, bo_ref, ln1w_ref, ln1b_ref,
                  w1_ref, b1_ref, w2_ref, b2_ref, ln2w_ref, ln2b_ref,
                  out_ref):
    out_ref[:] = _dense_pipeline(
        v_ref[:], None, wv_ref[:], bv_ref[:], wo_ref[:], bo_ref[:],
        ln1w_ref[:], ln1b_ref[:], w1_ref[:], b1_ref[:], w2_ref[:], b2_ref[:],
        ln2w_ref[:], ln2b_ref[:])


def _row_block(i):
    return (i, 0)


def _pinned(i):
    return (0, 0)


def _c1_block(i):
    return (i + _GRID, 0)


_W_SPECS = [
    pl.BlockSpec((_D, _D), _pinned),
    pl.BlockSpec((1, _D), _pinned),
    pl.BlockSpec((_D, _D), _pinned),
    pl.BlockSpec((1, _D), _pinned),
    pl.BlockSpec((1, _D), _pinned),
    pl.BlockSpec((1, _D), _pinned),
    pl.BlockSpec((_D, _D), _pinned),
    pl.BlockSpec((1, _D), _pinned),
    pl.BlockSpec((_D, _D), _pinned),
    pl.BlockSpec((1, _D), _pinned),
    pl.BlockSpec((1, _D), _pinned),
    pl.BlockSpec((1, _D), _pinned),
]

_tc_main = pl.pallas_call(
    _tc_main_body,
    grid=(_GRID,),
    in_specs=[pl.BlockSpec((_ROWS, _D), _row_block)] + _W_SPECS,
    out_specs=pl.BlockSpec((_ROWS, _D), _row_block),
    out_shape=jax.ShapeDtypeStruct((_N, _D), jnp.float32),
    compiler_params=pltpu.CompilerParams(
        dimension_semantics=("parallel",),
    ),
)


def _lane_to_rows(c, nrows):
    ct = c.T
    return jnp.concatenate(
        [lax.slice(ct, (0, a), (_D, a + 1)) for a in range(nrows // _D)],
        axis=0)


def _tc_fix_body(c_ref, wv_ref, bv_ref, wo_ref, bo_ref,
                 ln1w_ref, ln1b_ref, w1_ref, b1_ref, w2_ref, b2_ref,
                 ln2w_ref, ln2b_ref, v_any, outin_any, out_any, xv_scr, sem):
    cr = _NPAD // _D
    c = c_ref[0:cr, :] + c_ref[cr:2 * cr, :]

    weights = (wv_ref[:], bv_ref[:], wo_ref[:], bo_ref[:], ln1w_ref[:],
               ln1b_ref[:], w1_ref[:], b1_ref[:], w2_ref[:], b2_ref[:],
               ln2w_ref[:], ln2b_ref[:])

    rpb = _ROWS // _D
    for j in range(_GRID):
        cj = lax.slice(c, (j * rpb, 0), ((j + 1) * rpb, _D))
        nrows = _ROWS if j < _GRID - 1 else _TAIL

        @pl.when(jnp.any(cj <= 0.5))
        def _(cj=cj, j=j, nrows=nrows):
            cp = pltpu.make_async_copy(
                v_any.at[pl.ds(j * _ROWS, nrows), :],
                xv_scr.at[pl.ds(0, nrows), :], sem)
            cp.start()
            cp.wait()
            cnt = _lane_to_rows(cj, _ROWS)[:nrows]
            ind = jnp.where(cnt > 0.5, 1.0, 0.0)
            xv = xv_scr[pl.ds(0, nrows), :]
            xv_scr[pl.ds(0, nrows), :] = _dense_pipeline(xv, ind, *weights)
            cp = pltpu.make_async_copy(
                xv_scr.at[pl.ds(0, nrows), :],
                out_any.at[pl.ds(j * _ROWS, nrows), :], sem)
            cp.start()
            cp.wait()


_tc_fix = pl.pallas_call(
    _tc_fix_body,
    grid=(1,),
    in_specs=[
        pl.BlockSpec((_NC * _NPAD // _D, _D), _pinned),
    ] + _W_SPECS + [
        pl.BlockSpec(memory_space=pl.ANY),
        pl.BlockSpec(memory_space=pl.ANY),
    ],
    out_specs=pl.BlockSpec(memory_space=pl.ANY),
    out_shape=jax.ShapeDtypeStruct((_N, _D), jnp.float32),
    scratch_shapes=[
        pltpu.VMEM((_ROWS, _D), jnp.float32),
        pltpu.SemaphoreType.DMA,
    ],
    input_output_aliases={14: 0},
    compiler_params=pltpu.CompilerParams(
        dimension_semantics=("arbitrary",),
    ),
)


def kernel(q, k, v, self_indices, neighbor_indices, Wq, bq, Wk, bk, Wv, bv,
           Wo, bo, ln1_w, ln1_b, W1, b1, W2, b2, ln2_w, ln2_b):
    payload = jnp.concatenate([
        jnp.ones((_EPW,), jnp.float32),
        jnp.zeros((_CHUNK,), jnp.float32),
        jnp.concatenate([jnp.zeros((_PAD0,), jnp.float32),
                         jnp.ones((_CHUNK - _PAD0,), jnp.float32)]),
    ])
    counts = _sc_count()(self_indices, payload)
    counts = counts.reshape(_NC * _NPAD // _D, _D)
    row = lambda a: a.reshape(1, _D)
    weights = (Wv, row(bv), Wo, row(bo), row(ln1_w), row(ln1_b),
               W1, row(b1), W2, row(b2), row(ln2_w), row(ln2_b))
    main = _tc_main(v, *weights)
    return _tc_fix(counts, *weights, v, main)


# speedup vs baseline: 1.4250x; 1.0174x over previous
import functools

import jax
import jax.numpy as jnp
import numpy as np
from jax import lax
from jax.experimental import pallas as pl
from jax.experimental.pallas import tpu as pltpu
from jax.experimental.pallas import tpu_sc as plsc

_N = 10000
_E = 320000
_D = 128

_NC = 2
_NS = 16
_NW = _NC * _NS
_EPW = _E // _NW
_CHUNK = 640
_NPAD = _NS * _CHUNK
_PAD0 = _N - (_NS - 1) * _CHUNK

_ROWS = 2560

_PAYLOAD = np.concatenate([
    np.ones(_EPW, np.float32),
    np.zeros(_CHUNK, np.float32),
    np.concatenate([np.zeros(_PAD0, np.float32),
                    np.ones(_CHUNK - _PAD0, np.float32)]),
])
_GRID = _NPAD // _ROWS
_TAIL = _N - (_GRID - 1) * _ROWS


def _sc_count_body(idx_hbm, pay_hbm, out_hbm, idx_v, pay_v, shared,
                   sem_p, sem_i):
    cid = lax.axis_index("c")
    sid = lax.axis_index("s")
    wid = sid * _NC + cid

    cp_p = pltpu.async_copy(pay_hbm, pay_v, sem_p)
    cp_i = pltpu.async_copy(idx_hbm.at[pl.ds(wid * _EPW, _EPW)], idx_v, sem_i)
    cp_p.wait()
    init_off = jnp.where(sid == _NS - 1, _EPW + _CHUNK, _EPW)
    pltpu.sync_copy(pay_v.at[pl.ds(init_off, _CHUNK)],
                    shared.at[pl.ds(sid * _CHUNK, _CHUNK)])
    cp_i.wait()
    plsc.subcore_barrier()
    pltpu.sync_copy(pay_v.at[pl.ds(0, _EPW)], shared.at[idx_v], add=True)
    plsc.subcore_barrier()
    pltpu.sync_copy(
        shared.at[pl.ds(sid * _CHUNK, _CHUNK)],
        out_hbm.at[pl.ds(cid * _NPAD + sid * _CHUNK, _CHUNK)],
    )


@functools.cache
def _sc_count():
    return pl.kernel(
        _sc_count_body,
        out_type=jax.ShapeDtypeStruct((_NC * _NPAD,), jnp.float32),
        mesh=plsc.VectorSubcoreMesh(core_axis_name="c", subcore_axis_name="s",
                                    num_cores=_NC, num_subcores=_NS),
        scratch_types=[
            pltpu.VMEM((_EPW,), jnp.int32),
            pltpu.VMEM((_EPW + 2 * _CHUNK,), jnp.float32),
            pltpu.VMEM_SHARED((_NPAD,), jnp.float32),
            pltpu.SemaphoreType.DMA,
            pltpu.SemaphoreType.DMA,
        ],
    )


def _layernorm(x, w, b):
    m = jnp.mean(x, axis=-1, keepdims=True)
    var = jnp.mean((x - m) * (x - m), axis=-1, keepdims=True)
    return (x - m) * lax.rsqrt(var + 1e-5) * w + b


def _dot_nt(x, w):
    return lax.dot_general(x, w, (((1,), (1,)), ((), ())),
                           preferred_element_type=jnp.float32)


def _dense_pipeline(xv, ind, wv, bv, wo, bo, ln1w, ln1b, w1, b1, w2, b2,
                    ln2w, ln2b):
    vl = _dot_nt(xv, wv) + bv
    if ind is not None:
        vl = vl * ind
    v2 = _dot_nt(vl, wo) + bo
    x = _layernorm(xv + v2, ln1w, ln1b)
    h = jnp.maximum(_dot_nt(x, w1) + b1, 0.0)
    v2 = _dot_nt(h, w2) + b2
    return _layernorm(x + v2, ln2w, ln2b)


def _tc_main_body(v_ref, wv_ref, bv_ref, wo_ref, bo_ref, ln1w_ref, ln1b_ref,
                  w1_ref, b1_ref, w2_ref, b2_ref, ln2w_ref, ln2b_ref,
                  out_ref):
    out_ref[:] = _dense_pipeline(
        v_ref[:], None, wv_ref[:], bv_ref[:], wo_ref[:], bo_ref[:],
        ln1w_ref[:], ln1b_ref[:], w1_ref[:], b1_ref[:], w2_ref[:], b2_ref[:],
        ln2w_ref[:], ln2b_ref[:])


def _row_block(i):
    return (i, 0)


def _pinned(i):
    return (0, 0)


def _c1_block(i):
    return (i + _GRID, 0)


_W_SPECS = [
    pl.BlockSpec((_D, _D), _pinned),
    pl.BlockSpec((1, _D), _pinned),
    pl.BlockSpec((_D, _D), _pinned),
    pl.BlockSpec((1, _D), _pinned),
    pl.BlockSpec((1, _D), _pinned),
    pl.BlockSpec((1, _D), _pinned),
    pl.BlockSpec((_D, _D), _pinned),
    pl.BlockSpec((1, _D), _pinned),
    pl.BlockSpec((_D, _D), _pinned),
    pl.BlockSpec((1, _D), _pinned),
    pl.BlockSpec((1, _D), _pinned),
    pl.BlockSpec((1, _D), _pinned),
]

_tc_main = pl.pallas_call(
    _tc_main_body,
    grid=(_GRID,),
    in_specs=[pl.BlockSpec((_ROWS, _D), _row_block)] + _W_SPECS,
    out_specs=pl.BlockSpec((_ROWS, _D), _row_block),
    out_shape=jax.ShapeDtypeStruct((_N, _D), jnp.float32),
    compiler_params=pltpu.CompilerParams(
        dimension_semantics=("parallel",),
    ),
)


def _lane_to_rows(c, nrows):
    ct = c.T
    return jnp.concatenate(
        [lax.slice(ct, (0, a), (_D, a + 1)) for a in range(nrows // _D)],
        axis=0)


def _tc_fix_body(c_ref, wv_ref, bv_ref, wo_ref, bo_ref,
                 ln1w_ref, ln1b_ref, w1_ref, b1_ref, w2_ref, b2_ref,
                 ln2w_ref, ln2b_ref, v_any, outin_any, out_any, xv_scr, sem):
    cr = _NPAD // _D
    c = c_ref[0:cr, :] + c_ref[cr:2 * cr, :]

    weights = (wv_ref[:], bv_ref[:], wo_ref[:], bo_ref[:], ln1w_ref[:],
               ln1b_ref[:], w1_ref[:], b1_ref[:], w2_ref[:], b2_ref[:],
               ln2w_ref[:], ln2b_ref[:])

    rpb = _ROWS // _D
    for j in range(_GRID):
        cj = lax.slice(c, (j * rpb, 0), ((j + 1) * rpb, _D))
        nrows = _ROWS if j < _GRID - 1 else _TAIL

        @pl.when(jnp.any(cj <= 0.5))
        def _(cj=cj, j=j, nrows=nrows):
            cp = pltpu.make_async_copy(
                v_any.at[pl.ds(j * _ROWS, nrows), :],
                xv_scr.at[pl.ds(0, nrows), :], sem)
            cp.start()
            cp.wait()
            cnt = _lane_to_rows(cj, _ROWS)[:nrows]
            ind = jnp.where(cnt > 0.5, 1.0, 0.0)
            xv = xv_scr[pl.ds(0, nrows), :]
            xv_scr[pl.ds(0, nrows), :] = _dense_pipeline(xv, ind, *weights)
            cp = pltpu.make_async_copy(
                xv_scr.at[pl.ds(0, nrows), :],
                out_any.at[pl.ds(j * _ROWS, nrows), :], sem)
            cp.start()
            cp.wait()


_tc_fix = pl.pallas_call(
    _tc_fix_body,
    grid=(1,),
    in_specs=[
        pl.BlockSpec((_NC * _NPAD // _D, _D), _pinned),
    ] + _W_SPECS + [
        pl.BlockSpec(memory_space=pl.ANY),
        pl.BlockSpec(memory_space=pl.ANY),
    ],
    out_specs=pl.BlockSpec(memory_space=pl.ANY),
    out_shape=jax.ShapeDtypeStruct((_N, _D), jnp.float32),
    scratch_shapes=[
        pltpu.VMEM((_ROWS, _D), jnp.float32),
        pltpu.SemaphoreType.DMA,
    ],
    input_output_aliases={14: 0},
    compiler_params=pltpu.CompilerParams(
        dimension_semantics=("arbitrary",),
    ),
)


def kernel(q, k, v, self_indices, neighbor_indices, Wq, bq, Wk, bk, Wv, bv,
           Wo, bo, ln1_w, ln1_b, W1, b1, W2, b2, ln2_w, ln2_b):
    counts = _sc_count()(self_indices, jnp.asarray(_PAYLOAD))
    counts = counts.reshape(_NC * _NPAD // _D, _D)
    row = lambda a: a.reshape(1, _D)
    weights = (Wv, row(bv), Wo, row(bo), row(ln1_w), row(ln1_b),
               W1, row(b1), W2, row(b2), row(ln2_w), row(ln2_b))
    main = _tc_main(v, *weights)
    return _tc_fix(counts, *weights, v, main)


# speedup vs baseline: 1.4269x; 1.0013x over previous
import functools

import jax
import jax.numpy as jnp
import numpy as np
from jax import lax
from jax.experimental import pallas as pl
from jax.experimental.pallas import tpu as pltpu
from jax.experimental.pallas import tpu_sc as plsc

_N = 10000
_E = 320000
_D = 128

_NC = 2
_NS = 16
_NW = _NC * _NS
_EPW = _E // _NW
_CHUNK = 640
_NPAD = _NS * _CHUNK
_PAD0 = _N - (_NS - 1) * _CHUNK

_ROWS = 2560

_PAYLOAD = np.concatenate([
    np.ones(_EPW, np.float32),
    np.zeros(_CHUNK, np.float32),
    np.concatenate([np.zeros(_PAD0, np.float32),
                    np.ones(_CHUNK - _PAD0, np.float32)]),
])
_GRID = _NPAD // _ROWS
_TAIL = _N - (_GRID - 1) * _ROWS


def _sc_count_body(idx_hbm, pay_hbm, out_hbm, idx_v, pay_v, shared,
                   sem_p, sem_i):
    cid = lax.axis_index("c")
    sid = lax.axis_index("s")
    wid = sid * _NC + cid

    cp_p = pltpu.async_copy(pay_hbm, pay_v, sem_p)
    cp_i = pltpu.async_copy(idx_hbm.at[pl.ds(wid * _EPW, _EPW)], idx_v, sem_i)
    cp_p.wait()
    init_off = jnp.where(sid == _NS - 1, _EPW + _CHUNK, _EPW)
    pltpu.sync_copy(pay_v.at[pl.ds(init_off, _CHUNK)],
                    shared.at[pl.ds(sid * _CHUNK, _CHUNK)])
    cp_i.wait()
    plsc.subcore_barrier()
    pltpu.sync_copy(pay_v.at[pl.ds(0, _EPW)], shared.at[idx_v], add=True)
    plsc.subcore_barrier()
    pltpu.sync_copy(
        shared.at[pl.ds(sid * _CHUNK, _CHUNK)],
        out_hbm.at[pl.ds(cid * _NPAD + sid * _CHUNK, _CHUNK)],
    )


@functools.cache
def _sc_count():
    return pl.kernel(
        _sc_count_body,
        out_type=jax.ShapeDtypeStruct((_NC * _NPAD,), jnp.float32),
        mesh=plsc.VectorSubcoreMesh(core_axis_name="c", subcore_axis_name="s",
                                    num_cores=_NC, num_subcores=_NS),
        scratch_types=[
            pltpu.VMEM((_EPW,), jnp.int32),
            pltpu.VMEM((_EPW + 2 * _CHUNK,), jnp.float32),
            pltpu.VMEM_SHARED((_NPAD,), jnp.float32),
            pltpu.SemaphoreType.DMA,
            pltpu.SemaphoreType.DMA,
        ],
    )


def _layernorm(x, w, b):
    m = jnp.mean(x, axis=-1, keepdims=True)
    var = jnp.mean((x - m) * (x - m), axis=-1, keepdims=True)
    return (x - m) * lax.rsqrt(var + 1e-5) * w + b


def _dot_nt(x, w):
    return lax.dot_general(x, w, (((1,), (1,)), ((), ())),
                           preferred_element_type=jnp.float32)


def _dense_pipeline(xv, ind, wv, bv, wo, bo, ln1w, ln1b, w1, b1, w2, b2,
                    ln2w, ln2b):
    vl = _dot_nt(xv, wv) + bv
    if ind is not None:
        vl = vl * ind
    v2 = _dot_nt(vl, wo) + bo
    x = _layernorm(xv + v2, ln1w, ln1b)
    h = jnp.maximum(_dot_nt(x, w1) + b1, 0.0)
    v2 = _dot_nt(h, w2) + b2
    return _layernorm(x + v2, ln2w, ln2b)


def _tc_main_body(v_ref, wv_ref, bv_ref, wo_ref, bo_ref, ln1w_ref, ln1b_ref,
                  w1_ref, b1_ref, w2_ref, b2_ref, ln2w_ref, ln2b_ref,
                  out_ref):
    out_ref[:] = _dense_pipeline(
        v_ref[:], None, wv_ref[:], bv_ref[:], wo_ref[:], bo_ref[:],
        ln1w_ref[:], ln1b_ref[:], w1_ref[:], b1_ref[:], w2_ref[:], b2_ref[:],
        ln2w_ref[:], ln2b_ref[:])


def _row_block(i):
    return (i, 0)


def _pinned(i):
    return (0, 0)


_W_SPECS = [
    pl.BlockSpec((_D, _D), _pinned),
    pl.BlockSpec((1, _D), _pinned),
    pl.BlockSpec((_D, _D), _pinned),
    pl.BlockSpec((1, _D), _pinned),
    pl.BlockSpec((1, _D), _pinned),
    pl.BlockSpec((1, _D), _pinned),
    pl.BlockSpec((_D, _D), _pinned),
    pl.BlockSpec((1, _D), _pinned),
    pl.BlockSpec((_D, _D), _pinned),
    pl.BlockSpec((1, _D), _pinned),
    pl.BlockSpec((1, _D), _pinned),
    pl.BlockSpec((1, _D), _pinned),
]

_tc_main = pl.pallas_call(
    _tc_main_body,
    grid=(_GRID,),
    in_specs=[pl.BlockSpec((_ROWS, _D), _row_block)] + _W_SPECS,
    out_specs=pl.BlockSpec((_ROWS, _D), _row_block),
    out_shape=jax.ShapeDtypeStruct((_N, _D), jnp.float32),
    compiler_params=pltpu.CompilerParams(
        dimension_semantics=("parallel",),
    ),
)


def _lane_to_rows(c, nrows):
    ct = c.T
    return jnp.concatenate(
        [lax.slice(ct, (0, a), (_D, a + 1)) for a in range(nrows // _D)],
        axis=0)


def _tc_fix_body(c_ref, wv_ref, bv_ref, wo_ref, bo_ref,
                 ln1w_ref, ln1b_ref, w1_ref, b1_ref, w2_ref, b2_ref,
                 ln2w_ref, ln2b_ref, v_any, outin_any, out_any, xv_scr, sem):
    cr = _NPAD // _D
    c = c_ref[0:cr, :] + c_ref[cr:2 * cr, :]

    weights = (wv_ref[:], bv_ref[:], wo_ref[:], bo_ref[:], ln1w_ref[:],
               ln1b_ref[:], w1_ref[:], b1_ref[:], w2_ref[:], b2_ref[:],
               ln2w_ref[:], ln2b_ref[:])

    rpb = _ROWS // _D
    for j in range(_GRID):
        cj = lax.slice(c, (j * rpb, 0), ((j + 1) * rpb, _D))
        nrows = _ROWS if j < _GRID - 1 else _TAIL

        @pl.when(jnp.any(cj <= 0.5))
        def _(cj=cj, j=j, nrows=nrows):
            cp = pltpu.make_async_copy(
                v_any.at[pl.ds(j * _ROWS, nrows), :],
                xv_scr.at[pl.ds(0, nrows), :], sem)
            cp.start()
            cp.wait()
            cnt = _lane_to_rows(cj, _ROWS)[:nrows]
            ind = jnp.where(cnt > 0.5, 1.0, 0.0)
            xv = xv_scr[pl.ds(0, nrows), :]
            xv_scr[pl.ds(0, nrows), :] = _dense_pipeline(xv, ind, *weights)
            cp = pltpu.make_async_copy(
                xv_scr.at[pl.ds(0, nrows), :],
                out_any.at[pl.ds(j * _ROWS, nrows), :], sem)
            cp.start()
            cp.wait()


_tc_fix = pl.pallas_call(
    _tc_fix_body,
    grid=(1,),
    in_specs=[
        pl.BlockSpec((_NC * _NPAD // _D, _D), _pinned),
    ] + _W_SPECS + [
        pl.BlockSpec(memory_space=pl.ANY),
        pl.BlockSpec(memory_space=pl.ANY),
    ],
    out_specs=pl.BlockSpec(memory_space=pl.ANY),
    out_shape=jax.ShapeDtypeStruct((_N, _D), jnp.float32),
    scratch_shapes=[
        pltpu.VMEM((_ROWS, _D), jnp.float32),
        pltpu.SemaphoreType.DMA,
    ],
    input_output_aliases={14: 0},
    compiler_params=pltpu.CompilerParams(
        dimension_semantics=("arbitrary",),
    ),
)


def kernel(q, k, v, self_indices, neighbor_indices, Wq, bq, Wk, bk, Wv, bv,
           Wo, bo, ln1_w, ln1_b, W1, b1, W2, b2, ln2_w, ln2_b):
    counts = _sc_count()(self_indices, jnp.asarray(_PAYLOAD))
    counts = counts.reshape(_NC * _NPAD // _D, _D)
    row = lambda a: a.reshape(1, _D)
    weights = (Wv, row(bv), Wo, row(bo), row(ln1_w), row(ln1_b),
               W1, row(b1), W2, row(b2), row(ln2_w), row(ln2_b))
    main = _tc_main(v, *weights)
    return _tc_fix(counts, *weights, v, main)
